# Initial kernel scaffold; baseline (speedup 1.0000x reference)
import jax
import jax.numpy as jnp
from jax.experimental import pallas as pl


def kernel(op_feats, config_feats, emb_table, W, b, op_weights, config_weights, op_code):
    raise NotImplementedError("write your pallas kernel here")


# speedup vs baseline: 1.6786x; 1.6786x over previous
import functools

import jax
import jax.numpy as jnp
from jax import lax
from jax.experimental import pallas as pl
from jax.experimental.pallas import tpu as pltpu

N = 100000
NUM_EMB = 128
EMB_DIM = 128
OP_FEAT_DIM = 140
CFG_DIM = 18
OUT_DIM = 128
MAX_NORM = 1.0

BLOCK_N = 2000


def _prep_kernel(emb_ref, w2t_ref, opw_ref, t2_ref):
    rows = emb_ref[...]
    norms = jnp.sqrt(jnp.sum(rows * rows, axis=1, keepdims=True))
    scale = jnp.where(norms > MAX_NORM, MAX_NORM / (norms + 1e-7), 1.0)
    scaled = rows * (scale * opw_ref[0, 0])
    t2_ref[...] = jnp.dot(scaled, w2t_ref[...], preferred_element_type=jnp.float32)


def _main_kernel(opf_ref, cfg_ref, code_ref, w1t_ref, t2_ref, w3t_ref,
                 cfgw_ref, b_ref, out_ref):
    codes = code_ref[...]
    onehot = (codes == lax.broadcasted_iota(jnp.int32, (BLOCK_N, NUM_EMB), 1))
    onehot = onehot.astype(jnp.float32)
    acc = jnp.dot(opf_ref[...], w1t_ref[...], preferred_element_type=jnp.float32)
    acc = acc + jnp.dot(onehot, t2_ref[...], preferred_element_type=jnp.float32)
    acc = acc + jnp.dot(cfg_ref[...] * cfgw_ref[...], w3t_ref[...],
                        preferred_element_type=jnp.float32)
    out_ref[...] = acc + b_ref[...]


@functools.partial(jax.jit, static_argnames=("interpret",))
def _run(op_feats, config_feats, emb_table, W, b, op_weights, config_weights,
         op_code, interpret=False):
    w1t = W[:, :OP_FEAT_DIM].T
    w2t = W[:, OP_FEAT_DIM:OP_FEAT_DIM + EMB_DIM].T
    w3t = W[:, OP_FEAT_DIM + EMB_DIM:].T
    opw = op_weights.astype(jnp.float32).reshape(1, 1)
    cfgw = config_weights.astype(jnp.float32).reshape(1, CFG_DIM)
    codes = op_code.astype(jnp.int32).reshape(N, 1)
    b2 = b.reshape(1, OUT_DIM)

    t2 = pl.pallas_call(
        _prep_kernel,
        out_shape=jax.ShapeDtypeStruct((NUM_EMB, OUT_DIM), jnp.float32),
        interpret=interpret,
    )(emb_table, w2t, opw)

    grid = N // BLOCK_N
    out = pl.pallas_call(
        _main_kernel,
        grid=(grid,),
        in_specs=[
            pl.BlockSpec((BLOCK_N, OP_FEAT_DIM), lambda i: (i, 0)),
            pl.BlockSpec((BLOCK_N, CFG_DIM), lambda i: (i, 0)),
            pl.BlockSpec((BLOCK_N, 1), lambda i: (i, 0)),
            pl.BlockSpec((OP_FEAT_DIM, OUT_DIM), lambda i: (0, 0)),
            pl.BlockSpec((NUM_EMB, OUT_DIM), lambda i: (0, 0)),
            pl.BlockSpec((CFG_DIM, OUT_DIM), lambda i: (0, 0)),
            pl.BlockSpec((1, CFG_DIM), lambda i: (0, 0)),
            pl.BlockSpec((1, OUT_DIM), lambda i: (0, 0)),
        ],
        out_specs=pl.BlockSpec((BLOCK_N, OUT_DIM), lambda i: (i, 0)),
        out_shape=jax.ShapeDtypeStruct((N, OUT_DIM), jnp.float32),
        compiler_params=pltpu.CompilerParams(
            dimension_semantics=("parallel",),
        ),
        interpret=interpret,
    )(op_feats, config_feats, codes, w1t, t2, w3t, cfgw, b2)
    return out


def kernel(op_feats, config_feats, emb_table, W, b, op_weights, config_weights,
           op_code):
    return _run(op_feats, config_feats, emb_table, W, b, op_weights,
                config_weights, op_code)


# speedup vs baseline: 1.7520x; 1.0438x over previous
import functools

import jax
import jax.numpy as jnp
from jax import lax
from jax.experimental import pallas as pl
from jax.experimental.pallas import tpu as pltpu

N = 100000
NUM_EMB = 128
EMB_DIM = 128
OP_FEAT_DIM = 140
CFG_DIM = 18
OUT_DIM = 128
MAX_NORM = 1.0

BLOCK_N = 4000


def _prep_kernel(emb_ref, w2t_ref, opw_ref, t2_ref):
    rows = emb_ref[...]
    norms = jnp.sqrt(jnp.sum(rows * rows, axis=1, keepdims=True))
    scale = jnp.where(norms > MAX_NORM, MAX_NORM / (norms + 1e-7), 1.0)
    scaled = rows * (scale * opw_ref[0, 0])
    t2_ref[...] = jnp.dot(scaled, w2t_ref[...], preferred_element_type=jnp.float32)


def _main_kernel(opf_ref, cfg_ref, code_ref, w1t_ref, t2_ref, w3t_ref,
                 cfgw_ref, b_ref, out_ref):
    codes = code_ref[...]
    onehot = (codes == lax.broadcasted_iota(jnp.int32, (BLOCK_N, NUM_EMB), 1))
    onehot = onehot.astype(jnp.float32)
    acc = jnp.dot(opf_ref[...], w1t_ref[...], preferred_element_type=jnp.float32)
    acc = acc + jnp.dot(onehot, t2_ref[...], preferred_element_type=jnp.float32)
    acc = acc + jnp.dot(cfg_ref[...] * cfgw_ref[...], w3t_ref[...],
                        preferred_element_type=jnp.float32)
    out_ref[...] = acc + b_ref[...]


@functools.partial(jax.jit, static_argnames=("interpret",))
def _run(op_feats, config_feats, emb_table, W, b, op_weights, config_weights,
         op_code, interpret=False):
    w1t = W[:, :OP_FEAT_DIM].T
    w2t = W[:, OP_FEAT_DIM:OP_FEAT_DIM + EMB_DIM].T
    w3t = W[:, OP_FEAT_DIM + EMB_DIM:].T
    opw = op_weights.astype(jnp.float32).reshape(1, 1)
    cfgw = config_weights.astype(jnp.float32).reshape(1, CFG_DIM)
    codes = op_code.astype(jnp.int32).reshape(N, 1)
    b2 = b.reshape(1, OUT_DIM)

    t2 = pl.pallas_call(
        _prep_kernel,
        out_shape=jax.ShapeDtypeStruct((NUM_EMB, OUT_DIM), jnp.float32),
        interpret=interpret,
    )(emb_table, w2t, opw)

    grid = N // BLOCK_N
    out = pl.pallas_call(
        _main_kernel,
        grid=(grid,),
        in_specs=[
            pl.BlockSpec((BLOCK_N, OP_FEAT_DIM), lambda i: (i, 0)),
            pl.BlockSpec((BLOCK_N, CFG_DIM), lambda i: (i, 0)),
            pl.BlockSpec((BLOCK_N, 1), lambda i: (i, 0)),
            pl.BlockSpec((OP_FEAT_DIM, OUT_DIM), lambda i: (0, 0)),
            pl.BlockSpec((NUM_EMB, OUT_DIM), lambda i: (0, 0)),
            pl.BlockSpec((CFG_DIM, OUT_DIM), lambda i: (0, 0)),
            pl.BlockSpec((1, CFG_DIM), lambda i: (0, 0)),
            pl.BlockSpec((1, OUT_DIM), lambda i: (0, 0)),
        ],
        out_specs=pl.BlockSpec((BLOCK_N, OUT_DIM), lambda i: (i, 0)),
        out_shape=jax.ShapeDtypeStruct((N, OUT_DIM), jnp.float32),
        compiler_params=pltpu.CompilerParams(
            dimension_semantics=("parallel",),
        ),
        interpret=interpret,
    )(op_feats, config_feats, codes, w1t, t2, w3t, cfgw, b2)
    return out


def kernel(op_feats, config_feats, emb_table, W, b, op_weights, config_weights,
           op_code):
    return _run(op_feats, config_feats, emb_table, W, b, op_weights,
                config_weights, op_code)


# speedup vs baseline: 2.3212x; 1.3249x over previous
import functools

import jax
import jax.numpy as jnp
from jax import lax
from jax.experimental import pallas as pl
from jax.experimental.pallas import tpu as pltpu

N = 100000
NUM_EMB = 128
EMB_DIM = 128
OP_FEAT_DIM = 140
CFG_DIM = 18
OUT_DIM = 128
MAX_NORM = 1.0

BLOCK_N = 4096


def _prep_kernel(emb_ref, w2_ref, opw_ref, t2t_ref):
    rows = emb_ref[...]
    norms = jnp.sqrt(jnp.sum(rows * rows, axis=1, keepdims=True))
    scale = jnp.where(norms > MAX_NORM, MAX_NORM / (norms + 1e-7), 1.0)
    scaled = rows * (scale * opw_ref[0, 0])
    t2t_ref[...] = jnp.dot(w2_ref[...], scaled.T,
                           preferred_element_type=jnp.float32)


def _main_kernel(opf_ref, cfg_ref, code_ref, w1t_ref, t2t_ref, w3t_ref,
                 cfgw_ref, b_ref, out_ref):
    codes = code_ref[...]
    onehot_t = (codes[None, :] ==
                lax.broadcasted_iota(jnp.int32, (NUM_EMB, BLOCK_N), 0))
    emb_t = jnp.dot(t2t_ref[...], onehot_t.astype(jnp.float32),
                    preferred_element_type=jnp.float32)
    acc = jnp.dot(opf_ref[...], w1t_ref[...], preferred_element_type=jnp.float32)
    acc = acc + emb_t.T
    acc = acc + jnp.dot(cfg_ref[...] * cfgw_ref[...], w3t_ref[...],
                        preferred_element_type=jnp.float32)
    out_ref[...] = acc + b_ref[...]


@functools.partial(jax.jit, static_argnames=("interpret",))
def _run(op_feats, config_feats, emb_table, W, b, op_weights, config_weights,
         op_code, interpret=False):
    w1t = W[:, :OP_FEAT_DIM].T
    w2 = W[:, OP_FEAT_DIM:OP_FEAT_DIM + EMB_DIM]
    w3t = W[:, OP_FEAT_DIM + EMB_DIM:].T
    opw = op_weights.astype(jnp.float32).reshape(1, 1)
    cfgw = config_weights.astype(jnp.float32).reshape(1, CFG_DIM)
    codes = op_code.astype(jnp.int32).reshape(N)
    b2 = b.reshape(1, OUT_DIM)

    t2t = pl.pallas_call(
        _prep_kernel,
        out_shape=jax.ShapeDtypeStruct((OUT_DIM, NUM_EMB), jnp.float32),
        interpret=interpret,
    )(emb_table, w2, opw)

    grid = (N + BLOCK_N - 1) // BLOCK_N
    out = pl.pallas_call(
        _main_kernel,
        grid=(grid,),
        in_specs=[
            pl.BlockSpec((BLOCK_N, OP_FEAT_DIM), lambda i: (i, 0)),
            pl.BlockSpec((BLOCK_N, CFG_DIM), lambda i: (i, 0)),
            pl.BlockSpec((BLOCK_N,), lambda i: (i,)),
            pl.BlockSpec((OP_FEAT_DIM, OUT_DIM), lambda i: (0, 0)),
            pl.BlockSpec((OUT_DIM, NUM_EMB), lambda i: (0, 0)),
            pl.BlockSpec((CFG_DIM, OUT_DIM), lambda i: (0, 0)),
            pl.BlockSpec((1, CFG_DIM), lambda i: (0, 0)),
            pl.BlockSpec((1, OUT_DIM), lambda i: (0, 0)),
        ],
        out_specs=pl.BlockSpec((BLOCK_N, OUT_DIM), lambda i: (i, 0)),
        out_shape=jax.ShapeDtypeStruct((N, OUT_DIM), jnp.float32),
        compiler_params=pltpu.CompilerParams(
            dimension_semantics=("arbitrary",),
        ),
        interpret=interpret,
    )(op_feats, config_feats, codes, w1t, t2t, w3t, cfgw, b2)
    return out


def kernel(op_feats, config_feats, emb_table, W, b, op_weights, config_weights,
           op_code):
    return _run(op_feats, config_feats, emb_table, W, b, op_weights,
                config_weights, op_code)


# speedup vs baseline: 2.3525x; 1.0135x over previous
import functools

import jax
import jax.numpy as jnp
from jax import lax
from jax.experimental import pallas as pl
from jax.experimental.pallas import tpu as pltpu

N = 100000
NUM_EMB = 128
EMB_DIM = 128
OP_FEAT_DIM = 140
CFG_DIM = 18
OUT_DIM = 128
MAX_NORM = 1.0

BLOCK_N = 8192


def _main_kernel(opf_ref, cfg_ref, code_ref, emb_ref, w1t_ref, w2_ref,
                 w3t_ref, opw_ref, cfgw_ref, b_ref, out_ref):
    rows = emb_ref[...]
    norms = jnp.sqrt(jnp.sum(rows * rows, axis=1, keepdims=True))
    scale = jnp.where(norms > MAX_NORM, MAX_NORM / (norms + 1e-7), 1.0)
    scaled = rows * (scale * opw_ref[0, 0])
    t2t = jnp.dot(w2_ref[...], scaled.T,
                  preferred_element_type=jnp.float32)

    codes = code_ref[...]
    onehot_t = (codes[None, :] ==
                lax.broadcasted_iota(jnp.int32, (NUM_EMB, BLOCK_N), 0))
    emb_t = jnp.dot(t2t, onehot_t.astype(jnp.float32),
                    preferred_element_type=jnp.float32)
    acc = jnp.dot(opf_ref[...], w1t_ref[...], preferred_element_type=jnp.float32)
    acc = acc + emb_t.T
    acc = acc + jnp.dot(cfg_ref[...] * cfgw_ref[...], w3t_ref[...],
                        preferred_element_type=jnp.float32)
    out_ref[...] = acc + b_ref[...]


@functools.partial(jax.jit, static_argnames=("interpret",))
def _run(op_feats, config_feats, emb_table, W, b, op_weights, config_weights,
         op_code, interpret=False):
    w1t = W[:, :OP_FEAT_DIM].T
    w2 = W[:, OP_FEAT_DIM:OP_FEAT_DIM + EMB_DIM]
    w3t = W[:, OP_FEAT_DIM + EMB_DIM:].T
    opw = op_weights.astype(jnp.float32).reshape(1, 1)
    cfgw = config_weights.astype(jnp.float32).reshape(1, CFG_DIM)
    codes = op_code.astype(jnp.int32).reshape(N)
    b2 = b.reshape(1, OUT_DIM)

    grid = (N + BLOCK_N - 1) // BLOCK_N
    out = pl.pallas_call(
        _main_kernel,
        grid=(grid,),
        in_specs=[
            pl.BlockSpec((BLOCK_N, OP_FEAT_DIM), lambda i: (i, 0)),
            pl.BlockSpec((BLOCK_N, CFG_DIM), lambda i: (i, 0)),
            pl.BlockSpec((BLOCK_N,), lambda i: (i,)),
            pl.BlockSpec((NUM_EMB, EMB_DIM), lambda i: (0, 0)),
            pl.BlockSpec((OP_FEAT_DIM, OUT_DIM), lambda i: (0, 0)),
            pl.BlockSpec((OUT_DIM, EMB_DIM), lambda i: (0, 0)),
            pl.BlockSpec((CFG_DIM, OUT_DIM), lambda i: (0, 0)),
            pl.BlockSpec((1, 1), lambda i: (0, 0)),
            pl.BlockSpec((1, CFG_DIM), lambda i: (0, 0)),
            pl.BlockSpec((1, OUT_DIM), lambda i: (0, 0)),
        ],
        out_specs=pl.BlockSpec((BLOCK_N, OUT_DIM), lambda i: (i, 0)),
        out_shape=jax.ShapeDtypeStruct((N, OUT_DIM), jnp.float32),
        compiler_params=pltpu.CompilerParams(
            dimension_semantics=("arbitrary",),
        ),
        interpret=interpret,
    )(op_feats, config_feats, codes, emb_table, w1t, w2, w3t, opw, cfgw, b2)
    return out


def kernel(op_feats, config_feats, emb_table, W, b, op_weights, config_weights,
           op_code):
    return _run(op_feats, config_feats, emb_table, W, b, op_weights,
                config_weights, op_code)
